# baseline (speedup 1.0000x reference)
import functools

import jax
import jax.numpy as jnp
from jax import lax
from jax.experimental import pallas as pl
from jax.experimental.pallas import tpu as pltpu
from jax.experimental.pallas import tpu_sc as plsc

_VOCAB = 100000
_D = 1024
_B = 4
_S = 8192
_SCALE = 32.0
_EPS = 1e-5

_R = _S * _B
_NW = 32
_C = 16
_NBUF = 4
_NPIECE = 2

_RB = 2048


def _make_gather(nrows):
    rpw = nrows // _NW
    nch = rpw // _C

    def body(idx_hbm, tok_hbm, out_hbm,
             idx_v, buf_v,
             gsem0, gsem1, gsem2, gsem3, wsem0, wsem1, wsem2, wsem3):
        wid = lax.axis_index("s") * 2 + lax.axis_index("c")
        base = wid * rpw
        gsems = (gsem0, gsem1, gsem2, gsem3)
        wsems = (wsem0, wsem1, wsem2, wsem3)

        pltpu.sync_copy(idx_hbm.at[pl.ds(pl.multiple_of(base, rpw), rpw)],
                        idx_v)

        def start_fetch(g, b):
            off = pl.ds(pl.multiple_of(g * _C, _C), _C)
            pltpu.make_async_copy(
                tok_hbm.at[idx_v.at[off]], buf_v.at[b], gsems[b]).start()

        def wait_fetch(b):
            pltpu.make_async_copy(
                tok_hbm.at[idx_v.at[pl.ds(0, _C)]], buf_v.at[b],
                gsems[b]).wait()

        def start_wb(g, b):
            row0 = pl.multiple_of(base + g * _C, _C)
            pltpu.make_async_copy(
                buf_v.at[b], out_hbm.at[pl.ds(row0, _C)], wsems[b]).start()

        def wait_wb(b):
            pltpu.make_async_copy(
                buf_v.at[b], out_hbm.at[pl.ds(0, _C)], wsems[b]).wait()

        def chunk(g, b, wait_writeback, prefetch):
            wait_fetch(b)
            start_wb(g, b)
            b2 = (b + 2) % _NBUF
            if wait_writeback:
                wait_wb(b2)
            if prefetch:
                start_fetch(g + 2, b2)

        start_fetch(0, 0)
        start_fetch(1, 1)
        chunk(0, 0, wait_writeback=False, prefetch=True)
        chunk(1, 1, wait_writeback=False, prefetch=True)
        chunk(2, 2, wait_writeback=True, prefetch=True)
        chunk(3, 3, wait_writeback=True, prefetch=True)

        def round_body(i, _):
            g0 = i * _NBUF
            chunk(g0, 0, wait_writeback=True, prefetch=True)
            chunk(g0 + 1, 1, wait_writeback=True, prefetch=True)
            chunk(g0 + 2, 2, wait_writeback=True, prefetch=True)
            chunk(g0 + 3, 3, wait_writeback=True, prefetch=True)
            return 0

        lax.fori_loop(1, nch // _NBUF - 1, round_body, 0)
        chunk(nch - 4, 0, wait_writeback=True, prefetch=True)
        chunk(nch - 3, 1, wait_writeback=True, prefetch=True)
        chunk(nch - 2, 2, wait_writeback=False, prefetch=False)
        chunk(nch - 1, 3, wait_writeback=False, prefetch=False)
        wait_wb(0)
        wait_wb(1)
        wait_wb(2)
        wait_wb(3)

    return functools.partial(
        pl.kernel,
        mesh=plsc.VectorSubcoreMesh(core_axis_name="c", subcore_axis_name="s"),
        out_type=jax.ShapeDtypeStruct((nrows, _D), jnp.float32),
        scratch_types=[
            pltpu.VMEM((rpw,), jnp.int32),
            pltpu.VMEM((_NBUF, _C, _D), jnp.float32),
            pltpu.SemaphoreType.DMA,
            pltpu.SemaphoreType.DMA,
            pltpu.SemaphoreType.DMA,
            pltpu.SemaphoreType.DMA,
            pltpu.SemaphoreType.DMA,
            pltpu.SemaphoreType.DMA,
            pltpu.SemaphoreType.DMA,
            pltpu.SemaphoreType.DMA,
        ],
    )(body)


_gather_piece = _make_gather(_R // _NPIECE)


def _ln_body(g_ref, p_ref, gam_ref, bet_ref, o_ref):
    y = g_ref[...] * _SCALE + p_ref[...]
    mean = jnp.mean(y, axis=-1, keepdims=True)
    yc = y - mean
    var = jnp.mean(yc * yc, axis=-1, keepdims=True)
    o_ref[...] = yc * lax.rsqrt(var + _EPS) * gam_ref[0] + bet_ref[0]


_PB = _B // _NPIECE
_SBLK = _S // _RB


def _ln_body_passthrough(g_ref, p_ref, gam_ref, bet_ref, prev_ref, o_ref):
    del prev_ref
    _ln_body(g_ref, p_ref, gam_ref, bet_ref, o_ref)


def _make_ln_piece(piece):
    first = piece == 0
    in_specs = [
        pl.BlockSpec((_RB, _D),
                     lambda i: ((i % _PB) * _SBLK + i // _PB, 0)),
        pl.BlockSpec((_RB, _D), lambda i: (i // _PB, 0)),
        pl.BlockSpec((1, _D), lambda i: (0, 0)),
        pl.BlockSpec((1, _D), lambda i: (0, 0)),
    ]
    if not first:
        in_specs.append(pl.BlockSpec(memory_space=pltpu.MemorySpace.HBM))
    return pl.pallas_call(
        _ln_body if first else _ln_body_passthrough,
        grid=(_S * _PB // _RB,),
        in_specs=in_specs,
        out_specs=pl.BlockSpec(
            (_RB, _D), lambda i: (i // _PB, piece * _PB + i % _PB)),
        out_shape=jax.ShapeDtypeStruct((_S, _B * _D), jnp.float32),
        input_output_aliases={} if first else {4: 0},
        compiler_params=pltpu.CompilerParams(
            dimension_semantics=("parallel",)),
    )


_ln_pieces = [_make_ln_piece(p) for p in range(_NPIECE)]


def kernel(src_tokens, tok_table, pos_table, ln_gamma, ln_beta):
    idx = src_tokens.reshape(-1)
    gam = ln_gamma.reshape(1, _D)
    bet = ln_beta.reshape(1, _D)
    rp = _R // _NPIECE
    gathered = [_gather_piece(idx[p * rp:(p + 1) * rp], tok_table)
                for p in range(_NPIECE)]
    out = _ln_pieces[0](gathered[0], pos_table, gam, bet)
    for p in range(1, _NPIECE):
        out = _ln_pieces[p](gathered[p], pos_table, gam, bet, out)
    return out.reshape(_S, _B, _D)


# speedup vs baseline: 1.0338x; 1.0338x over previous
import functools

import jax
import jax.numpy as jnp
from jax import lax
from jax.experimental import pallas as pl
from jax.experimental.pallas import tpu as pltpu
from jax.experimental.pallas import tpu_sc as plsc

_VOCAB = 100000
_D = 1024
_B = 4
_S = 8192
_SCALE = 32.0
_EPS = 1e-5

_R = _S * _B
_NW = 32
_C = 16
_NBUF = 4
_NPIECE = 2

_RB = 2048


def _make_gather(nrows):
    rpw = nrows // _NW
    nch = rpw // _C

    def body(idx_hbm, tok_hbm, out_hbm,
             idx_v, buf_v,
             gsem0, gsem1, gsem2, gsem3, wsem0, wsem1, wsem2, wsem3):
        wid = lax.axis_index("s") * 2 + lax.axis_index("c")
        base = wid * rpw
        gsems = (gsem0, gsem1, gsem2, gsem3)
        wsems = (wsem0, wsem1, wsem2, wsem3)

        pltpu.sync_copy(idx_hbm.at[pl.ds(pl.multiple_of(base, rpw), rpw)],
                        idx_v)

        def start_fetch(g, b):
            off = pl.ds(pl.multiple_of(g * _C, _C), _C)
            pltpu.make_async_copy(
                tok_hbm.at[idx_v.at[off]], buf_v.at[b], gsems[b]).start()

        def wait_fetch(b):
            pltpu.make_async_copy(
                tok_hbm.at[idx_v.at[pl.ds(0, _C)]], buf_v.at[b],
                gsems[b]).wait()

        def start_wb(g, b):
            row0 = pl.multiple_of(base + g * _C, _C)
            pltpu.make_async_copy(
                buf_v.at[b], out_hbm.at[pl.ds(row0, _C)], wsems[b]).start()

        def wait_wb(b):
            pltpu.make_async_copy(
                buf_v.at[b], out_hbm.at[pl.ds(0, _C)], wsems[b]).wait()

        def chunk(g, b, wait_writeback, prefetch):
            wait_fetch(b)
            start_wb(g, b)
            b2 = (b + 2) % _NBUF
            if wait_writeback:
                wait_wb(b2)
            if prefetch:
                start_fetch(g + 2, b2)

        start_fetch(0, 0)
        start_fetch(1, 1)
        chunk(0, 0, wait_writeback=False, prefetch=True)
        chunk(1, 1, wait_writeback=False, prefetch=True)
        chunk(2, 2, wait_writeback=True, prefetch=True)
        chunk(3, 3, wait_writeback=True, prefetch=True)

        def round_body(i, _):
            g0 = i * _NBUF
            chunk(g0, 0, wait_writeback=True, prefetch=True)
            chunk(g0 + 1, 1, wait_writeback=True, prefetch=True)
            chunk(g0 + 2, 2, wait_writeback=True, prefetch=True)
            chunk(g0 + 3, 3, wait_writeback=True, prefetch=True)
            return 0

        lax.fori_loop(1, nch // _NBUF - 1, round_body, 0)
        chunk(nch - 4, 0, wait_writeback=True, prefetch=True)
        chunk(nch - 3, 1, wait_writeback=True, prefetch=True)
        chunk(nch - 2, 2, wait_writeback=False, prefetch=False)
        chunk(nch - 1, 3, wait_writeback=False, prefetch=False)
        wait_wb(0)
        wait_wb(1)
        wait_wb(2)
        wait_wb(3)

    return functools.partial(
        pl.kernel,
        mesh=plsc.VectorSubcoreMesh(core_axis_name="c", subcore_axis_name="s"),
        out_type=jax.ShapeDtypeStruct((nrows, _D), jnp.float32),
        scratch_types=[
            pltpu.VMEM((rpw,), jnp.int32),
            pltpu.VMEM((_NBUF, _C, _D), jnp.float32),
            pltpu.SemaphoreType.DMA,
            pltpu.SemaphoreType.DMA,
            pltpu.SemaphoreType.DMA,
            pltpu.SemaphoreType.DMA,
            pltpu.SemaphoreType.DMA,
            pltpu.SemaphoreType.DMA,
            pltpu.SemaphoreType.DMA,
            pltpu.SemaphoreType.DMA,
        ],
    )(body)


_gather_piece = _make_gather(_R // _NPIECE)


def _ln_body(g_ref, p_ref, gam_ref, bet_ref, o_ref):
    y = g_ref[...] * _SCALE + p_ref[...]
    mean = jnp.mean(y, axis=-1, keepdims=True)
    yc = y - mean
    var = jnp.mean(yc * yc, axis=-1, keepdims=True)
    o_ref[...] = yc * lax.rsqrt(var + _EPS) * gam_ref[0] + bet_ref[0]


_SP = _S // _NPIECE
_PSBLK = _SP // _RB


def _ln_body_passthrough(g_ref, p_ref, gam_ref, bet_ref, prev_ref, o_ref):
    del prev_ref
    _ln_body(g_ref, p_ref, gam_ref, bet_ref, o_ref)


def _make_ln_piece(piece):
    first = piece == 0
    in_specs = [
        pl.BlockSpec((_RB, _D),
                     lambda i: ((i % _B) * _PSBLK + i // _B, 0)),
        pl.BlockSpec((_RB, _D), lambda i: (piece * _PSBLK + i // _B, 0)),
        pl.BlockSpec((1, _D), lambda i: (0, 0)),
        pl.BlockSpec((1, _D), lambda i: (0, 0)),
    ]
    if not first:
        in_specs.append(pl.BlockSpec(memory_space=pltpu.MemorySpace.HBM))
    return pl.pallas_call(
        _ln_body if first else _ln_body_passthrough,
        grid=(_SP * _B // _RB,),
        in_specs=in_specs,
        out_specs=pl.BlockSpec(
            (_RB, _D), lambda i: (piece * _PSBLK + i // _B, i % _B)),
        out_shape=jax.ShapeDtypeStruct((_S, _B * _D), jnp.float32),
        input_output_aliases={} if first else {4: 0},
        compiler_params=pltpu.CompilerParams(
            dimension_semantics=("parallel",)),
    )


_ln_pieces = [_make_ln_piece(p) for p in range(_NPIECE)]


def kernel(src_tokens, tok_table, pos_table, ln_gamma, ln_beta):
    gam = ln_gamma.reshape(1, _D)
    bet = ln_beta.reshape(1, _D)
    gathered = [
        _gather_piece(src_tokens[:, p * _SP:(p + 1) * _SP].reshape(-1),
                      tok_table)
        for p in range(_NPIECE)
    ]
    out = _ln_pieces[0](gathered[0], pos_table, gam, bet)
    for p in range(1, _NPIECE):
        out = _ln_pieces[p](gathered[p], pos_table, gam, bet, out)
    return out.reshape(_S, _B, _D)
